# baseline (speedup 1.0000x reference)
import functools

import jax
import jax.numpy as jnp
from jax import lax
from jax.experimental import pallas as pl
from jax.experimental.pallas import tpu as pltpu
from jax.experimental.pallas import tpu_sc as plsc

N = 10000
E = 320000
D = 128

NC = 2
NS = 16
NW = NC * NS
CHUNK = 128
NBW = 2 * ((E + NW * 2 * CHUNK - 1) // (NW * 2 * CHUNK))
EPW = NBW * CHUNK
E_PAD = EPW * NW
ACC_ROWS = 10240
ZROWS = 64
ROWS_PER_TILE = ACC_ROWS // NS


def _spmm_sc(h, eidx):
  mesh = plsc.VectorSubcoreMesh(core_axis_name="c", subcore_axis_name="s")

  @functools.partial(
      pl.kernel,
      out_type=jax.ShapeDtypeStruct((NC, ACC_ROWS, D), jnp.float32),
      mesh=mesh,
      scratch_types=[
          pltpu.VMEM((2, CHUNK), jnp.int32),
          pltpu.VMEM((2, CHUNK), jnp.int32),
          pltpu.VMEM((CHUNK, D), jnp.float32),
          pltpu.VMEM((CHUNK, D), jnp.float32),
          pltpu.VMEM((ZROWS, D), jnp.float32),
          pltpu.VMEM_SHARED((ACC_ROWS, D), jnp.float32),
          pltpu.SemaphoreType.DMA,
          pltpu.SemaphoreType.DMA,
      ],
  )
  def k(h_hbm, eidx_hbm, out_hbm, idx0, idx1, rows0, rows1, zbuf, acc,
        sem0, sem1):
    cid = lax.axis_index("c")
    sid = lax.axis_index("s")

    zero = jnp.zeros((16,), jnp.float32)

    def zrow(i, _):
      zbuf[i // (D // 16), pl.ds((i % (D // 16)) * 16, 16)] = zero
      return 0

    lax.fori_loop(0, ZROWS * (D // 16), zrow, 0)

    def zacc(j, _):
      pltpu.sync_copy(zbuf, acc.at[pl.ds(sid * ROWS_PER_TILE + j * ZROWS, ZROWS)])
      return 0

    lax.fori_loop(0, ROWS_PER_TILE // ZROWS, zacc, 0)
    plsc.subcore_barrier()

    wid = sid * NC + cid
    bblk = wid * NBW

    pltpu.sync_copy(eidx_hbm.at[bblk], idx0)
    pltpu.async_copy(h_hbm.at[idx0.at[0]], rows0, sem0)

    def body(j, _):
      pltpu.sync_copy(eidx_hbm.at[bblk + 2 * j + 1], idx1)
      pltpu.async_copy(h_hbm.at[idx1.at[0]], rows1, sem1)
      pltpu.make_async_copy(h_hbm.at[pl.ds(0, CHUNK)], rows0, sem0).wait()
      pltpu.sync_copy(rows0, acc.at[idx0.at[1]], add=True)

      @pl.when(j < NBW // 2 - 1)
      def _():
        pltpu.sync_copy(eidx_hbm.at[bblk + 2 * j + 2], idx0)
        pltpu.async_copy(h_hbm.at[idx0.at[0]], rows0, sem0)

      pltpu.make_async_copy(h_hbm.at[pl.ds(0, CHUNK)], rows1, sem1).wait()
      pltpu.sync_copy(rows1, acc.at[idx1.at[1]], add=True)
      return 0

    lax.fori_loop(0, NBW // 2, body, 0)
    plsc.subcore_barrier()

    pltpu.sync_copy(acc.at[pl.ds(sid * ROWS_PER_TILE, ROWS_PER_TILE)],
                    out_hbm.at[cid, pl.ds(sid * ROWS_PER_TILE, ROWS_PER_TILE)])

  return k(h, eidx)


_BLK = 1000


def _lin1_body(x_ref, w_ref, b_ref, o_ref):
  o_ref[...] = lax.dot_general(
      x_ref[...], w_ref[...], (((1,), (1,)), ((), ())),
      preferred_element_type=jnp.float32) + b_ref[...]


def _lin2_body(p0_ref, p1_ref, w_ref, b_ref, o_ref):
  z = jnp.maximum(p0_ref[...] + p1_ref[...], 0.0)
  o_ref[...] = lax.dot_general(
      z, w_ref[...], (((1,), (1,)), ((), ())),
      preferred_element_type=jnp.float32) + b_ref[...]


def _add_body(a_ref, b_ref, o_ref):
  o_ref[...] = a_ref[...] + b_ref[...]


def _linear1(x, W, b):
  return pl.pallas_call(
      _lin1_body,
      grid=(N // _BLK,),
      in_specs=[
          pl.BlockSpec((_BLK, D), lambda i: (i, 0)),
          pl.BlockSpec((D, D), lambda i: (0, 0)),
          pl.BlockSpec((1, D), lambda i: (0, 0)),
      ],
      out_specs=pl.BlockSpec((_BLK, D), lambda i: (i, 0)),
      out_shape=jax.ShapeDtypeStruct((N, D), jnp.float32),
  )(x, W, b)


def _linear2(p0, p1, W, b):
  return pl.pallas_call(
      _lin2_body,
      grid=(N // _BLK,),
      in_specs=[
          pl.BlockSpec((_BLK, D), lambda i: (i, 0)),
          pl.BlockSpec((_BLK, D), lambda i: (i, 0)),
          pl.BlockSpec((D, D), lambda i: (0, 0)),
          pl.BlockSpec((1, D), lambda i: (0, 0)),
      ],
      out_specs=pl.BlockSpec((_BLK, D), lambda i: (i, 0)),
      out_shape=jax.ShapeDtypeStruct((N, D), jnp.float32),
  )(p0, p1, W, b)


def _add(a, b):
  return pl.pallas_call(
      _add_body,
      grid=(N // _BLK,),
      in_specs=[
          pl.BlockSpec((_BLK, D), lambda i: (i, 0)),
          pl.BlockSpec((_BLK, D), lambda i: (i, 0)),
      ],
      out_specs=pl.BlockSpec((_BLK, D), lambda i: (i, 0)),
      out_shape=jax.ShapeDtypeStruct((N, D), jnp.float32),
  )(a, b)


def kernel(x, edge_index, W1, b1, W2, b2):
  dst = edge_index[0]
  src = edge_index[1]
  pad = E_PAD - E
  src_p = jnp.concatenate([src, jnp.zeros((pad,), jnp.int32)])
  dummy_dst = N + jnp.arange(pad, dtype=jnp.int32) % (ACC_ROWS - N)
  dst_p = jnp.concatenate([dst, dummy_dst])
  eidx = jnp.stack([src_p.reshape(-1, CHUNK), dst_p.reshape(-1, CHUNK)], axis=1)
  b1r = b1.reshape(1, D)
  b2r = b2.reshape(1, D)

  h1 = _linear1(x, W1, b1r)
  P1 = _spmm_sc(h1, eidx)
  h2 = _linear2(P1[0], P1[1], W2, b2r)
  P2 = _spmm_sc(h2, eidx)
  return _add(P2[0], P2[1])
